# baseline (speedup 1.0000x reference)
import math

import jax
import jax.numpy as jnp
from jax import lax
from jax.experimental import pallas as pl
from jax.experimental.pallas import tpu as pltpu
from jax.experimental.pallas import tpu_sc as plsc

VOCAB = 100000
EMB = 128
B = 1024
L = 200
SCALE = math.sqrt(EMB)

_INFO = plsc.get_sparse_core_info()
NC, NS, LANES = _INFO.num_cores, _INFO.num_subcores, _INFO.num_lanes
NW = NC * NS

N_TOK = B * L
PER_W = N_TOK // NW
CHUNK = 128
N_CHUNKS = PER_W // CHUNK
GRP = 2
SUPER = GRP * CHUNK
N_STAGES = N_CHUNKS // GRP
NBUF = 3


def _body(tokens_hbm, table_hbm, out_hbm, idx_v, bufs, sems_g, sems_s):
    wid = lax.axis_index("s") * NC + lax.axis_index("c")
    base = wid * PER_W
    pltpu.sync_copy(tokens_hbm.at[wid], idx_v)

    def start_gathers(s, slot):
        for h in range(GRP):
            pltpu.async_copy(
                table_hbm.at[idx_v.at[s * GRP + h]],
                bufs.at[slot, pl.ds(h * CHUNK, CHUNK)],
                sems_g.at[slot],
            )

    def wait_gathers(slot):
        for h in range(GRP):
            pltpu.make_async_copy(
                table_hbm.at[pl.ds(0, CHUNK)],
                bufs.at[slot, pl.ds(h * CHUNK, CHUNK)],
                sems_g.at[slot],
            ).wait()

    def start_scatter(s, slot):
        pltpu.async_copy(
            bufs.at[slot], out_hbm.at[pl.ds(base + s * SUPER, SUPER)],
            sems_s.at[slot],
        )

    def wait_scatter(slot):
        pltpu.make_async_copy(
            bufs.at[slot], out_hbm.at[pl.ds(base, SUPER)], sems_s.at[slot]
        ).wait()

    def scale(slot):
        buf = bufs.at[slot]

        @pl.loop(0, SUPER, unroll=4)
        def _row(r):
            for j in range(EMB // LANES):
                buf[r, pl.ds(j * LANES, LANES)] = (
                    buf[r, pl.ds(j * LANES, LANES)] * SCALE
                )

    def stage(s, slot, prefetch, drain):
        nxt = (slot + 1) % NBUF
        if drain:
            wait_scatter(nxt)
        if prefetch:
            start_gathers(s + 1, nxt)
        wait_gathers(slot)
        scale(slot)
        start_scatter(s, slot)

    start_gathers(0, 0)

    for b in range(NBUF):
        stage(b, b, prefetch=True, drain=(b + 1 >= NBUF))

    @pl.loop(NBUF, N_STAGES - 1, step=NBUF)
    def _block(c):
        for b in range(NBUF):
            stage(c + b, b, prefetch=True, drain=True)

    stage(N_STAGES - 1, (N_STAGES - 1) % NBUF, prefetch=False, drain=False)

    for b in range(NBUF):
        wait_scatter(b)


@jax.jit
def _embed(tokens_grouped, table):
    kfn = pl.kernel(
        _body,
        out_type=jax.ShapeDtypeStruct((N_TOK, EMB), jnp.float32),
        mesh=plsc.VectorSubcoreMesh(core_axis_name="c", subcore_axis_name="s"),
        scratch_types=[
            pltpu.VMEM((N_CHUNKS, CHUNK), jnp.int32),
            pltpu.VMEM((NBUF, SUPER, EMB), jnp.float32),
            pltpu.SemaphoreType.DMA((NBUF,)),
            pltpu.SemaphoreType.DMA((NBUF,)),
        ],
    )
    return kfn(tokens_grouped, table)


def kernel(tokens, table):
    tokens_grouped = tokens.reshape(NW, N_CHUNKS, CHUNK).astype(jnp.int32)
    out = _embed(tokens_grouped, table)
    return out.reshape(B, L, EMB)


# speedup vs baseline: 1.0005x; 1.0005x over previous
import math

import jax
import jax.numpy as jnp
from jax import lax
from jax.experimental import pallas as pl
from jax.experimental.pallas import tpu as pltpu
from jax.experimental.pallas import tpu_sc as plsc

VOCAB = 100000
EMB = 128
B = 1024
L = 200
SCALE = math.sqrt(EMB)

_INFO = plsc.get_sparse_core_info()
NC, NS, LANES = _INFO.num_cores, _INFO.num_subcores, _INFO.num_lanes
NW = NC * NS

N_TOK = B * L
PER_W = N_TOK // NW
CHUNK = 128
N_CHUNKS = PER_W // CHUNK
GRP = 2
SUPER = GRP * CHUNK
N_STAGES = N_CHUNKS // GRP
NBUF = 3


def _body(tokens_hbm, table_hbm, out_hbm, idx_v, bufs, sems_g, sems_s):
    wid = lax.axis_index("s") * NC + lax.axis_index("c")
    base = wid * PER_W
    pltpu.sync_copy(tokens_hbm.at[wid], idx_v)

    def start_gathers(s, slot):
        for h in range(GRP):
            pltpu.async_copy(
                table_hbm.at[idx_v.at[s * GRP + h]],
                bufs.at[slot, pl.ds(h * CHUNK, CHUNK)],
                sems_g.at[slot],
            )

    def wait_gathers(slot):
        for h in range(GRP):
            pltpu.make_async_copy(
                table_hbm.at[pl.ds(0, CHUNK)],
                bufs.at[slot, pl.ds(h * CHUNK, CHUNK)],
                sems_g.at[slot],
            ).wait()

    def start_scatter(s, slot):
        pltpu.async_copy(
            bufs.at[slot], out_hbm.at[pl.ds(base + s * SUPER, SUPER)],
            sems_s.at[slot],
        )

    def wait_scatter(slot):
        pltpu.make_async_copy(
            bufs.at[slot], out_hbm.at[pl.ds(base, SUPER)], sems_s.at[slot]
        ).wait()

    def scale(slot):
        buf = bufs.at[slot]

        @plsc.parallel_loop(0, SUPER, unroll=4)
        def _row(r):
            for j in range(EMB // LANES):
                buf[r, pl.ds(j * LANES, LANES)] = (
                    buf[r, pl.ds(j * LANES, LANES)] * SCALE
                )

    def stage(s, slot, prefetch, drain):
        nxt = (slot + 1) % NBUF
        if drain:
            wait_scatter(nxt)
        if prefetch:
            start_gathers(s + 1, nxt)
        wait_gathers(slot)
        scale(slot)
        start_scatter(s, slot)

    start_gathers(0, 0)

    for b in range(NBUF):
        stage(b, b, prefetch=True, drain=(b + 1 >= NBUF))

    @pl.loop(NBUF, N_STAGES - 1, step=NBUF)
    def _block(c):
        for b in range(NBUF):
            stage(c + b, b, prefetch=True, drain=True)

    stage(N_STAGES - 1, (N_STAGES - 1) % NBUF, prefetch=False, drain=False)

    for b in range(NBUF):
        wait_scatter(b)


@jax.jit
def _embed(tokens_grouped, table):
    kfn = pl.kernel(
        _body,
        out_type=jax.ShapeDtypeStruct((N_TOK, EMB), jnp.float32),
        mesh=plsc.VectorSubcoreMesh(core_axis_name="c", subcore_axis_name="s"),
        scratch_types=[
            pltpu.VMEM((N_CHUNKS, CHUNK), jnp.int32),
            pltpu.VMEM((NBUF, SUPER, EMB), jnp.float32),
            pltpu.SemaphoreType.DMA((NBUF,)),
            pltpu.SemaphoreType.DMA((NBUF,)),
        ],
    )
    return kfn(tokens_grouped, table)


def kernel(tokens, table):
    tokens_grouped = tokens.reshape(NW, N_CHUNKS, CHUNK).astype(jnp.int32)
    out = _embed(tokens_grouped, table)
    return out.reshape(B, L, EMB)
